# Initial kernel scaffold; baseline (speedup 1.0000x reference)
import jax
import jax.numpy as jnp
from jax.experimental import pallas as pl


def kernel(indexes, embedding_weight):
    raise NotImplementedError("write your pallas kernel here")


# speedup vs baseline: 1.5608x; 1.5608x over previous
import functools

import jax
import jax.numpy as jnp
from jax import lax
from jax.experimental import pallas as pl
from jax.experimental.pallas import tpu as pltpu
from jax.experimental.pallas import tpu_sc as plsc

NUM = 1000000
DIM = 32
BATCH = 16384
FIELDS = 26
B_TOTAL = BATCH * FIELDS

_info = plsc.get_sparse_core_info()
_NC, _NS = _info.num_cores, _info.num_subcores
_NW = _NC * _NS
_BPW = B_TOTAL // _NW
_CHUNK = 1664
_NCH = _BPW // _CHUNK

_mesh = plsc.VectorSubcoreMesh(core_axis_name="c", subcore_axis_name="s")


@functools.partial(
    pl.kernel,
    mesh=_mesh,
    out_type=jax.ShapeDtypeStruct((B_TOTAL, DIM), jnp.float32),
    scratch_types=[
        pltpu.VMEM((_CHUNK,), jnp.int32),
        pltpu.VMEM((_CHUNK, DIM), jnp.float32),
        pltpu.SemaphoreType.DMA,
    ],
    compiler_params=pltpu.CompilerParams(use_tc_tiling_on_sc=False),
)
def _sc_gather(idx_hbm, table_hbm, out_hbm, idx_v, rows_v, sem):
    wid = lax.axis_index("s") * _NC + lax.axis_index("c")
    base = wid * _BPW
    for c in range(_NCH):
        off = base + c * _CHUNK
        pltpu.sync_copy(idx_hbm.at[pl.ds(off, _CHUNK)], idx_v)
        pltpu.async_copy(table_hbm.at[idx_v], rows_v, sem).wait()
        pltpu.sync_copy(rows_v, out_hbm.at[pl.ds(off, _CHUNK)])


def kernel(indexes, embedding_weight):
    idx_flat = indexes.reshape(-1).astype(jnp.int32)
    out = _sc_gather(idx_flat, embedding_weight)
    return out.reshape(BATCH, FIELDS, DIM)


# speedup vs baseline: 1.5751x; 1.0092x over previous
import functools

import jax
import jax.numpy as jnp
from jax import lax
from jax.experimental import pallas as pl
from jax.experimental.pallas import tpu as pltpu
from jax.experimental.pallas import tpu_sc as plsc

NUM = 1000000
DIM = 32
BATCH = 16384
FIELDS = 26
B_TOTAL = BATCH * FIELDS

_info = plsc.get_sparse_core_info()
_NC, _NS = _info.num_cores, _info.num_subcores
_NW = _NC * _NS
_BPW = B_TOTAL // _NW
_CHUNK = 1664
_NCH = _BPW // _CHUNK

_mesh = plsc.VectorSubcoreMesh(core_axis_name="c", subcore_axis_name="s")


@functools.partial(
    pl.kernel,
    mesh=_mesh,
    out_type=jax.ShapeDtypeStruct((B_TOTAL, DIM), jnp.float32),
    scratch_types=[
        pltpu.VMEM((_CHUNK,), jnp.int32),
        pltpu.VMEM((_CHUNK,), jnp.int32),
        pltpu.VMEM((_CHUNK, DIM), jnp.float32),
        pltpu.VMEM((_CHUNK, DIM), jnp.float32),
        pltpu.SemaphoreType.DMA,
        pltpu.SemaphoreType.DMA,
        pltpu.SemaphoreType.DMA,
        pltpu.SemaphoreType.DMA,
        pltpu.SemaphoreType.DMA,
        pltpu.SemaphoreType.DMA,
    ],
    compiler_params=pltpu.CompilerParams(use_tc_tiling_on_sc=False),
)
def _sc_gather(idx_hbm, table_hbm, out_hbm, ib0, ib1, rb0, rb1,
               si0, si1, sg0, sg1, so0, so1):
    wid = lax.axis_index("s") * _NC + lax.axis_index("c")
    base = wid * _BPW
    ib, rb = [ib0, ib1], [rb0, rb1]
    si, sg, so = [si0, si1], [sg0, sg1], [so0, so1]
    idx_cp = [None] * _NCH
    g_cp = [None] * _NCH
    o_cp = [None] * _NCH

    def start_idx(c):
        idx_cp[c] = pltpu.async_copy(
            idx_hbm.at[pl.ds(base + c * _CHUNK, _CHUNK)], ib[c % 2], si[c % 2])

    start_idx(0)
    start_idx(1)
    for c in range(_NCH):
        b = c % 2
        idx_cp[c].wait()
        if c >= 2:
            o_cp[c - 2].wait()
        g_cp[c] = pltpu.async_copy(table_hbm.at[ib[b]], rb[b], sg[b])
        if c >= 1:
            pb = (c - 1) % 2
            g_cp[c - 1].wait()
            o_cp[c - 1] = pltpu.async_copy(
                rb[pb], out_hbm.at[pl.ds(base + (c - 1) * _CHUNK, _CHUNK)],
                so[pb])
            if c + 1 < _NCH:
                start_idx(c + 1)
    lb = (_NCH - 1) % 2
    g_cp[_NCH - 1].wait()
    o_cp[_NCH - 1] = pltpu.async_copy(
        rb[lb], out_hbm.at[pl.ds(base + (_NCH - 1) * _CHUNK, _CHUNK)], so[lb])
    o_cp[_NCH - 2].wait()
    o_cp[_NCH - 1].wait()


def kernel(indexes, embedding_weight):
    idx_flat = indexes.reshape(-1).astype(jnp.int32)
    out = _sc_gather(idx_flat, embedding_weight)
    return out.reshape(BATCH, FIELDS, DIM)
